# baseline (speedup 1.0000x reference)
import functools

import jax
import jax.numpy as jnp
from jax import lax
from jax.experimental import pallas as pl
from jax.experimental.pallas import tpu as pltpu
from jax.experimental.pallas import tpu_sc as plsc

N = 10000
NP = 10240
D = 128
E = 320000
NC = 2
NS = 16
CH = 128
NCHUNK = E // CH
PER_CORE = NCHUNK // NC
PT = 80
NB = 2
IB = 16
LAG = 8
EPAD = NC * NS * PT * CH
RPT = NP // NS

_MESH = dict(core_axis_name="c", subcore_axis_name="s", num_cores=NC,
             num_subcores=NS)


def _writeback(acc_sh, out_hbm, c, s):
    off = pl.multiple_of(c * NP + s * RPT, 8)
    pltpu.sync_copy(acc_sh.at[pl.ds(s * RPT, RPT)],
                    out_hbm.at[pl.ds(off, RPT)])


def _deg_body(dst_hbm, ones_hbm, zeros_hbm, out_hbm, ones_v, dsts_v,
              acc_sh, sem):
    c = lax.axis_index("c")
    s = lax.axis_index("s")
    cbase = pl.multiple_of((c * NS + s) * PT, 8)
    cnt = jnp.minimum(PT, NCHUNK - (c * NS + s) * PT)
    pltpu.sync_copy(zeros_hbm, acc_sh.at[pl.ds(s * RPT, RPT)])
    pltpu.sync_copy(ones_hbm, ones_v)
    pltpu.sync_copy(dst_hbm.at[pl.ds(cbase, PT)], dsts_v)
    plsc.subcore_barrier()

    def body(tt, carry):
        @pl.when(tt < cnt)
        def _():
            pltpu.async_copy(ones_v, acc_sh.at[dsts_v.at[tt]], sem, add=True)

        @pl.when(tt - LAG >= 0)
        def _():
            @pl.when(tt - LAG < cnt)
            def _():
                pltpu.make_async_copy(ones_v, acc_sh.at[dsts_v.at[tt]],
                                      sem).wait()
        return carry

    lax.fori_loop(0, PT, body, 0)

    def drain(tt, carry):
        @pl.when(tt < cnt)
        def _():
            pltpu.make_async_copy(ones_v, acc_sh.at[dsts_v.at[tt]],
                                  sem).wait()
        return carry

    lax.fori_loop(PT - LAG, PT, drain, 0)
    plsc.subcore_barrier()
    _writeback(acc_sh, out_hbm, c, s)


def _agg_body(y_hbm, src_hbm, dst_hbm, zeros_hbm, out_hbm, srcs_b, dsts_b,
              rows0, rows1, acc_sh, sg0, sg1, ss0, ss1):
    c = lax.axis_index("c")
    s = lax.axis_index("s")
    cbase = pl.multiple_of((c * NS + s) * PT, 8)
    cnt = jnp.minimum(PT, NCHUNK - (c * NS + s) * PT)
    rows = (rows0, rows1)
    sg = (sg0, sg1)
    ss = (ss0, ss1)

    def iband(u):
        return jnp.bitwise_and(u, 2 * IB - 1)

    def load_batch(tt0):
        half = jnp.bitwise_and(tt0, IB)
        off = pl.multiple_of(cbase + tt0, 8)
        pltpu.sync_copy(src_hbm.at[pl.ds(off, IB)],
                        srcs_b.at[pl.ds(half, IB)])
        pltpu.sync_copy(dst_hbm.at[pl.ds(off, IB)],
                        dsts_b.at[pl.ds(half, IB)])

    def gather_start(u, b):
        pltpu.async_copy(y_hbm.at[srcs_b.at[iband(u)]], rows[b], sg[b])

    pltpu.sync_copy(zeros_hbm, acc_sh.at[pl.ds(s * RPT, RPT)])
    load_batch(0)
    plsc.subcore_barrier()
    gather_start(0, 0)

    def body(jj, carry):
        for b in range(NB):
            u = NB * jj + b
            b2 = 1 - b

            @pl.when(u < cnt)
            def _(b=b, u=u):
                pltpu.make_async_copy(y_hbm.at[srcs_b.at[iband(u)]],
                                     rows[b], sg[b]).wait()
                pltpu.async_copy(rows[b], acc_sh.at[dsts_b.at[iband(u)]],
                                ss[b], add=True)

            @pl.when((u - 1 >= 0) & (u - 1 < cnt))
            def _(b2=b2, u=u):
                pltpu.make_async_copy(rows[b2],
                                      acc_sh.at[dsts_b.at[iband(u - 1)]],
                                      ss[b2]).wait()

            @pl.when(u + 1 < cnt)
            def _(b2=b2, u=u):
                gather_start(u + 1, b2)

        tt = NB * jj
        @pl.when((jnp.bitwise_and(tt, IB - 1) == 0) & (tt + IB < cnt))
        def _(tt=tt):
            load_batch(tt + IB)
        return carry

    lax.fori_loop(0, PT // NB + 1, body, 0)
    plsc.subcore_barrier()
    _writeback(acc_sh, out_hbm, c, s)


def _deg_call(dst2d, ones_i, zeros_i):
    mesh = plsc.VectorSubcoreMesh(**_MESH)
    f = pl.kernel(
        _deg_body,
        out_type=jax.ShapeDtypeStruct((2 * NP, D), jnp.int32),
        mesh=mesh,
        scratch_types=[
            pltpu.VMEM((CH, D), jnp.int32),
            pltpu.VMEM((PT, CH), jnp.int32),
            pltpu.VMEM_SHARED((NP, D), jnp.int32),
            pltpu.SemaphoreType.DMA,
        ],
    )
    return f(dst2d, ones_i, zeros_i)


def _agg_call(y, src2d, dst2d, zeros_f):
    mesh = plsc.VectorSubcoreMesh(**_MESH)
    f = pl.kernel(
        _agg_body,
        out_type=jax.ShapeDtypeStruct((2 * NP, D), jnp.float32),
        mesh=mesh,
        scratch_types=[
            pltpu.VMEM((2 * IB, CH), jnp.int32),
            pltpu.VMEM((2 * IB, CH), jnp.int32),
            pltpu.VMEM((CH, D), jnp.float32),
            pltpu.VMEM((CH, D), jnp.float32),
            pltpu.VMEM_SHARED((NP, D), jnp.float32),
            pltpu.SemaphoreType.DMA,
            pltpu.SemaphoreType.DMA,
            pltpu.SemaphoreType.DMA,
            pltpu.SemaphoreType.DMA,
        ],
    )
    return f(y, src2d, dst2d, zeros_f)


R = 640
G = NP // R
GN = -(-N // R)


def _dinv(d0, d1):
    deg = (d0[:, 0:1] + d1[:, 0:1] + 1).astype(jnp.float32)
    return lax.rsqrt(deg)


def _b_body(x_ref, w_ref, d0, d1, y_ref):
    xw = jnp.dot(x_ref[:, :], w_ref[:, :], preferred_element_type=jnp.float32)
    y_ref[:, :] = xw * _dinv(d0, d1)


def _d_body(p0, p1, y1_ref, d0, d1, b1_ref, w2_ref, h1_ref, y2_ref):
    dinv = _dinv(d0, d1)
    h1 = jnp.maximum(dinv * (p0[:, :] + p1[:, :] + y1_ref[:, :])
                     + b1_ref[:, :], 0.0)
    h1_ref[:, :] = h1
    y2_ref[:, :] = dinv * jnp.dot(h1, w2_ref[:, :],
                                  preferred_element_type=jnp.float32)


def _e_body(p0, p1, y2_ref, d0, d1, b2_ref, h1_ref, wl_ref, bl_ref, o_ref):
    dinv = _dinv(d0, d1)
    h2 = jnp.maximum(dinv * (p0[:, :] + p1[:, :] + y2_ref[:, :])
                     + b2_ref[:, :], 0.0)
    o_ref[:, :] = (jnp.dot(h1_ref[:, :], wl_ref[0:D, :],
                           preferred_element_type=jnp.float32)
                   + jnp.dot(h2, wl_ref[D:2 * D, :],
                             preferred_element_type=jnp.float32)
                   + bl_ref[:, :])


_ROW = pl.BlockSpec((R, D), lambda i: (i, 0))
_ROW1 = pl.BlockSpec((R, D), lambda i: (i + G, 0))
_DEG = pl.BlockSpec((R, D), lambda i: (i, 0))
_DEG1 = pl.BlockSpec((R, D), lambda i: (i + G, 0))
_WFULL = pl.BlockSpec((D, D), lambda i: (0, 0))
_BIAS = pl.BlockSpec((1, D), lambda i: (0, 0))


def _b_call(x, W1, degp):
    return pl.pallas_call(
        _b_body,
        grid=(GN,),
        in_specs=[_ROW, _WFULL, _DEG, _DEG1],
        out_specs=_ROW,
        out_shape=jax.ShapeDtypeStruct((N, D), jnp.float32),
    )(x, W1, degp, degp)


def _d_call(p1, y1, degp, b1, W2):
    return pl.pallas_call(
        _d_body,
        grid=(GN,),
        in_specs=[_ROW, _ROW1, _ROW, _DEG, _DEG1, _BIAS, _WFULL],
        out_specs=[_ROW, _ROW],
        out_shape=[jax.ShapeDtypeStruct((N, D), jnp.float32),
                   jax.ShapeDtypeStruct((N, D), jnp.float32)],
    )(p1, p1, y1, degp, degp, b1, W2)


def _e_call(p2, y2, degp, b2, h1, Wl, bl):
    return pl.pallas_call(
        _e_body,
        grid=(GN,),
        in_specs=[_ROW, _ROW1, _ROW, _DEG, _DEG1, _BIAS, _ROW,
                  pl.BlockSpec((2 * D, D), lambda i: (0, 0)), _BIAS],
        out_specs=_ROW,
        out_shape=jax.ShapeDtypeStruct((N, D), jnp.float32),
    )(p2, p2, y2, degp, degp, b2, h1, Wl, bl)


def kernel(x, edge_index, W1, b1, W2, b2, Wl, bl):
    ei = edge_index.astype(jnp.int32)
    pad = jnp.zeros((2, EPAD - E), jnp.int32)
    ei = jnp.concatenate([ei, pad], axis=1)
    src2d = ei[0].reshape(EPAD // CH, CH)
    dst2d = ei[1].reshape(EPAD // CH, CH)
    zeros_f = jnp.zeros((RPT, D), jnp.float32)
    zeros_i = jnp.zeros((RPT, D), jnp.int32)
    ones_i = jnp.ones((CH, D), jnp.int32)

    degp = _deg_call(dst2d, ones_i, zeros_i)
    y1 = _b_call(x, W1, degp)
    p1 = _agg_call(y1, src2d, dst2d, zeros_f)
    h1, y2 = _d_call(p1, y1, degp, b1.reshape(1, D), W2)
    p2 = _agg_call(y2, src2d, dst2d, zeros_f)
    return _e_call(p2, y2, degp, b2.reshape(1, D), h1, Wl, bl.reshape(1, D))


# speedup vs baseline: 1.1126x; 1.1126x over previous
import functools

import jax
import jax.numpy as jnp
from jax import lax
from jax.experimental import pallas as pl
from jax.experimental.pallas import tpu as pltpu
from jax.experimental.pallas import tpu_sc as plsc

N = 10000
NP = 10240
D = 128
E = 320000
NC = 2
NS = 16
CH = 128
NCHUNK = E // CH
PER_CORE = NCHUNK // NC
PT = 80
NB = 2
IB = 16
LAG = 12
EPAD = NC * NS * PT * CH
RPT = NP // NS

_MESH = dict(core_axis_name="c", subcore_axis_name="s", num_cores=NC,
             num_subcores=NS)


def _writeback(acc_sh, out_hbm, c, s):
    off = pl.multiple_of(c * NP + s * RPT, 8)
    pltpu.sync_copy(acc_sh.at[pl.ds(s * RPT, RPT)],
                    out_hbm.at[pl.ds(off, RPT)])


def _deg_body(dst_hbm, ones_hbm, zeros_hbm, out_hbm, ones_v, dsts_v,
              acc_sh, sem):
    c = lax.axis_index("c")
    s = lax.axis_index("s")
    cbase = pl.multiple_of((c * NS + s) * PT, 8)
    cnt = jnp.minimum(PT, NCHUNK - (c * NS + s) * PT)
    pltpu.sync_copy(zeros_hbm, acc_sh.at[pl.ds(s * RPT, RPT)])
    pltpu.sync_copy(ones_hbm, ones_v)
    pltpu.sync_copy(dst_hbm.at[pl.ds(cbase, PT)], dsts_v)
    plsc.subcore_barrier()

    def body(tt, carry):
        @pl.when(tt < cnt)
        def _():
            pltpu.async_copy(ones_v, acc_sh.at[dsts_v.at[tt]], sem, add=True)

        @pl.when(tt - LAG >= 0)
        def _():
            @pl.when(tt - LAG < cnt)
            def _():
                pltpu.make_async_copy(ones_v, acc_sh.at[dsts_v.at[tt]],
                                      sem).wait()
        return carry

    lax.fori_loop(0, PT, body, 0)

    def drain(tt, carry):
        @pl.when(tt < cnt)
        def _():
            pltpu.make_async_copy(ones_v, acc_sh.at[dsts_v.at[tt]],
                                  sem).wait()
        return carry

    lax.fori_loop(PT - LAG, PT, drain, 0)
    plsc.subcore_barrier()
    _writeback(acc_sh, out_hbm, c, s)


def _agg_body(y_hbm, src_hbm, dst_hbm, zeros_hbm, out_hbm, srcs_b, dsts_b,
              rows0, rows1, acc_sh, sg0, sg1):
    c = lax.axis_index("c")
    s = lax.axis_index("s")
    cbase = pl.multiple_of((c * NS + s) * PT, 8)
    cnt = jnp.minimum(PT, NCHUNK - (c * NS + s) * PT)
    rows = (rows0, rows1)
    sems = (sg0, sg1)

    def iband(u):
        return jnp.bitwise_and(u, 2 * IB - 1)

    def load_batch(tt0):
        half = jnp.bitwise_and(tt0, IB)
        off = pl.multiple_of(cbase + tt0, 8)
        pltpu.sync_copy(src_hbm.at[pl.ds(off, IB)],
                        srcs_b.at[pl.ds(half, IB)])
        pltpu.sync_copy(dst_hbm.at[pl.ds(off, IB)],
                        dsts_b.at[pl.ds(half, IB)])

    def gather_start(u, b):
        pltpu.async_copy(y_hbm.at[srcs_b.at[iband(u)]], rows[b], sems[b])

    pltpu.sync_copy(zeros_hbm, acc_sh.at[pl.ds(s * RPT, RPT)])
    load_batch(0)
    plsc.subcore_barrier()
    for b in range(NB):
        @pl.when(b < cnt)
        def _(b=b):
            gather_start(b, b)

    def body(jj, carry):
        tt = NB * jj

        @pl.when((jnp.bitwise_and(tt, IB - 1) == 0) & (tt + IB < cnt))
        def _():
            load_batch(tt + IB)

        for b in range(NB):
            u = tt + b

            @pl.when(u < cnt)
            def _(b=b, u=u):
                pltpu.make_async_copy(y_hbm.at[srcs_b.at[iband(u)]],
                                      rows[b], sems[b]).wait()
                pltpu.sync_copy(rows[b], acc_sh.at[dsts_b.at[iband(u)]],
                                add=True)

            @pl.when(u + NB < cnt)
            def _(b=b, u=u):
                gather_start(u + NB, b)
        return carry

    lax.fori_loop(0, PT // NB, body, 0)
    plsc.subcore_barrier()
    _writeback(acc_sh, out_hbm, c, s)


def _deg_call(dst2d, ones_i, zeros_i):
    mesh = plsc.VectorSubcoreMesh(**_MESH)
    f = pl.kernel(
        _deg_body,
        out_type=jax.ShapeDtypeStruct((2 * NP, D), jnp.int32),
        mesh=mesh,
        scratch_types=[
            pltpu.VMEM((CH, D), jnp.int32),
            pltpu.VMEM((PT, CH), jnp.int32),
            pltpu.VMEM_SHARED((NP, D), jnp.int32),
            pltpu.SemaphoreType.DMA,
        ],
    )
    return f(dst2d, ones_i, zeros_i)


def _agg_call(y, src2d, dst2d, zeros_f):
    mesh = plsc.VectorSubcoreMesh(**_MESH)
    f = pl.kernel(
        _agg_body,
        out_type=jax.ShapeDtypeStruct((2 * NP, D), jnp.float32),
        mesh=mesh,
        scratch_types=[
            pltpu.VMEM((2 * IB, CH), jnp.int32),
            pltpu.VMEM((2 * IB, CH), jnp.int32),
            pltpu.VMEM((CH, D), jnp.float32),
            pltpu.VMEM((CH, D), jnp.float32),
            pltpu.VMEM_SHARED((NP, D), jnp.float32),
            pltpu.SemaphoreType.DMA,
            pltpu.SemaphoreType.DMA,
        ],
    )
    return f(y, src2d, dst2d, zeros_f)


R = 640
G = NP // R
GN = -(-N // R)


def _dinv(d0, d1):
    deg = (d0[:, 0:1] + d1[:, 0:1] + 1).astype(jnp.float32)
    return lax.rsqrt(deg)


def _b_body(x_ref, w_ref, d0, d1, y_ref, dv_ref):
    dinv = _dinv(d0, d1)
    xw = jnp.dot(x_ref[:, :], w_ref[:, :], preferred_element_type=jnp.float32)
    y_ref[:, :] = xw * dinv
    dv_ref[:, :] = dinv


def _d_body(p0, p1, y1_ref, dv_ref, b1_ref, w2_ref, h1_ref, y2_ref):
    dinv = dv_ref[:, :]
    h1 = jnp.maximum(dinv * (p0[:, :] + p1[:, :] + y1_ref[:, :])
                     + b1_ref[:, :], 0.0)
    h1_ref[:, :] = h1
    y2_ref[:, :] = dinv * jnp.dot(h1, w2_ref[:, :],
                                  preferred_element_type=jnp.float32)


def _e_body(p0, p1, y2_ref, dv_ref, b2_ref, h1_ref, wl_ref, bl_ref, o_ref):
    dinv = dv_ref[:, :]
    h2 = jnp.maximum(dinv * (p0[:, :] + p1[:, :] + y2_ref[:, :])
                     + b2_ref[:, :], 0.0)
    o_ref[:, :] = (jnp.dot(h1_ref[:, :], wl_ref[0:D, :],
                           preferred_element_type=jnp.float32)
                   + jnp.dot(h2, wl_ref[D:2 * D, :],
                             preferred_element_type=jnp.float32)
                   + bl_ref[:, :])


_ROW = pl.BlockSpec((R, D), lambda i: (i, 0))
_ROW1 = pl.BlockSpec((R, D), lambda i: (i + G, 0))
_DEG = pl.BlockSpec((R, D), lambda i: (i, 0))
_DEG1 = pl.BlockSpec((R, D), lambda i: (i + G, 0))
_WFULL = pl.BlockSpec((D, D), lambda i: (0, 0))
_BIAS = pl.BlockSpec((1, D), lambda i: (0, 0))


_DV = pl.BlockSpec((R, 1), lambda i: (i, 0))


def _b_call(x, W1, degp):
    return pl.pallas_call(
        _b_body,
        grid=(GN,),
        in_specs=[_ROW, _WFULL, _DEG, _DEG1],
        out_specs=[_ROW, _DV],
        out_shape=[jax.ShapeDtypeStruct((N, D), jnp.float32),
                   jax.ShapeDtypeStruct((N, 1), jnp.float32)],
    )(x, W1, degp, degp)


def _d_call(p1, y1, dv, b1, W2):
    return pl.pallas_call(
        _d_body,
        grid=(GN,),
        in_specs=[_ROW, _ROW1, _ROW, _DV, _BIAS, _WFULL],
        out_specs=[_ROW, _ROW],
        out_shape=[jax.ShapeDtypeStruct((N, D), jnp.float32),
                   jax.ShapeDtypeStruct((N, D), jnp.float32)],
    )(p1, p1, y1, dv, b1, W2)


def _e_call(p2, y2, dv, b2, h1, Wl, bl):
    return pl.pallas_call(
        _e_body,
        grid=(GN,),
        in_specs=[_ROW, _ROW1, _ROW, _DV, _BIAS, _ROW,
                  pl.BlockSpec((2 * D, D), lambda i: (0, 0)), _BIAS],
        out_specs=_ROW,
        out_shape=jax.ShapeDtypeStruct((N, D), jnp.float32),
    )(p2, p2, y2, dv, b2, h1, Wl, bl)


def kernel(x, edge_index, W1, b1, W2, b2, Wl, bl):
    ei = edge_index.astype(jnp.int32)
    pad = jnp.zeros((2, EPAD - E), jnp.int32)
    ei = jnp.concatenate([ei, pad], axis=1)
    src2d = ei[0].reshape(EPAD // CH, CH)
    dst2d = ei[1].reshape(EPAD // CH, CH)
    zeros_f = jnp.zeros((RPT, D), jnp.float32)
    zeros_i = jnp.zeros((RPT, D), jnp.int32)
    ones_i = jnp.ones((CH, D), jnp.int32)

    degp = _deg_call(dst2d, ones_i, zeros_i)
    y1, dv = _b_call(x, W1, degp)
    p1 = _agg_call(y1, src2d, dst2d, zeros_f)
    h1, y2 = _d_call(p1, y1, dv, b1.reshape(1, D), W2)
    p2 = _agg_call(y2, src2d, dst2d, zeros_f)
    return _e_call(p2, y2, dv, b2.reshape(1, D), h1, Wl, bl.reshape(1, D))


# speedup vs baseline: 1.1317x; 1.0172x over previous
import functools

import jax
import jax.numpy as jnp
from jax import lax
from jax.experimental import pallas as pl
from jax.experimental.pallas import tpu as pltpu
from jax.experimental.pallas import tpu_sc as plsc

N = 10000
NP = 10240
D = 128
E = 320000
NC = 2
NS = 16
CH = 128
NCHUNK = E // CH
PER_CORE = NCHUNK // NC
PT = 80
NB = 2
IB = 16
LAG = 12
EPAD = NC * NS * PT * CH
RPT = NP // NS

_MESH = dict(core_axis_name="c", subcore_axis_name="s", num_cores=NC,
             num_subcores=NS)


def _writeback(acc_sh, out_hbm, c, s):
    off = pl.multiple_of(c * NP + s * RPT, 8)
    pltpu.sync_copy(acc_sh.at[pl.ds(s * RPT, RPT)],
                    out_hbm.at[pl.ds(off, RPT)])


def _deg_body(dst_hbm, ones_hbm, zeros_hbm, out_hbm, ones_v, dsts_v,
              acc_sh, sem):
    c = lax.axis_index("c")
    s = lax.axis_index("s")
    cbase = pl.multiple_of((c * NS + s) * PT, 8)
    cnt = jnp.minimum(PT, NCHUNK - (c * NS + s) * PT)
    pltpu.sync_copy(zeros_hbm, acc_sh.at[pl.ds(s * RPT, RPT)])
    pltpu.sync_copy(ones_hbm, ones_v)
    pltpu.sync_copy(dst_hbm.at[pl.ds(cbase, PT)], dsts_v)
    plsc.subcore_barrier()

    def body(tt, carry):
        @pl.when(tt < cnt)
        def _():
            pltpu.async_copy(ones_v, acc_sh.at[dsts_v.at[tt]], sem, add=True)

        @pl.when(tt - LAG >= 0)
        def _():
            @pl.when(tt - LAG < cnt)
            def _():
                pltpu.make_async_copy(ones_v, acc_sh.at[dsts_v.at[tt]],
                                      sem).wait()
        return carry

    lax.fori_loop(0, PT, body, 0)

    def drain(tt, carry):
        @pl.when(tt < cnt)
        def _():
            pltpu.make_async_copy(ones_v, acc_sh.at[dsts_v.at[tt]],
                                  sem).wait()
        return carry

    lax.fori_loop(PT - LAG, PT, drain, 0)
    plsc.subcore_barrier()
    _writeback(acc_sh, out_hbm, c, s)


def _agg_body(y_hbm, src_hbm, dst_hbm, zeros_hbm, out_hbm, srcs_b, dsts_b,
              rows0, rows1, acc_sh, sg0, sg1, si):
    c = lax.axis_index("c")
    s = lax.axis_index("s")
    cbase = pl.multiple_of((c * NS + s) * PT, 8)
    cnt = jnp.minimum(PT, NCHUNK - (c * NS + s) * PT)
    rows = (rows0, rows1)
    sems = (sg0, sg1)

    def iband(u):
        return jnp.bitwise_and(u, 2 * IB - 1)

    def refill_start(tt0):
        half = jnp.bitwise_and(tt0, IB)
        off = pl.multiple_of(cbase + tt0, 8)
        pltpu.async_copy(src_hbm.at[pl.ds(off, IB)],
                         srcs_b.at[pl.ds(half, IB)], si)
        pltpu.async_copy(dst_hbm.at[pl.ds(off, IB)],
                         dsts_b.at[pl.ds(half, IB)], si)

    def refill_wait(tt0):
        half = jnp.bitwise_and(tt0, IB)
        off = pl.multiple_of(cbase + tt0, 8)
        pltpu.make_async_copy(src_hbm.at[pl.ds(off, IB)],
                              srcs_b.at[pl.ds(half, IB)], si).wait()
        pltpu.make_async_copy(dst_hbm.at[pl.ds(off, IB)],
                              dsts_b.at[pl.ds(half, IB)], si).wait()

    def gather_start(u, b):
        pltpu.async_copy(y_hbm.at[srcs_b.at[iband(u)]], rows[b], sems[b])

    pltpu.sync_copy(zeros_hbm, acc_sh.at[pl.ds(s * RPT, RPT)])
    refill_start(0)
    refill_wait(0)
    plsc.subcore_barrier()
    for b in range(NB):
        @pl.when(b < cnt)
        def _(b=b):
            gather_start(b, b)

    def body(jj, carry):
        tt = NB * jj

        @pl.when((jnp.bitwise_and(tt, IB - 1) == 0) & (tt + IB < cnt))
        def _():
            refill_start(tt + IB)

        @pl.when((jnp.bitwise_and(tt, IB - 1) == IB - 2) & (tt + 2 < cnt))
        def _():
            refill_wait(tt + 2)

        for b in range(NB):
            u = tt + b

            @pl.when(u < cnt)
            def _(b=b, u=u):
                pltpu.make_async_copy(y_hbm.at[srcs_b.at[iband(u)]],
                                      rows[b], sems[b]).wait()
                pltpu.sync_copy(rows[b], acc_sh.at[dsts_b.at[iband(u)]],
                                add=True)

            @pl.when(u + NB < cnt)
            def _(b=b, u=u):
                gather_start(u + NB, b)
        return carry

    lax.fori_loop(0, PT // NB, body, 0)
    plsc.subcore_barrier()
    _writeback(acc_sh, out_hbm, c, s)


def _deg_call(dst2d, ones_i, zeros_i):
    mesh = plsc.VectorSubcoreMesh(**_MESH)
    f = pl.kernel(
        _deg_body,
        out_type=jax.ShapeDtypeStruct((2 * NP, D), jnp.int32),
        mesh=mesh,
        scratch_types=[
            pltpu.VMEM((CH, D), jnp.int32),
            pltpu.VMEM((PT, CH), jnp.int32),
            pltpu.VMEM_SHARED((NP, D), jnp.int32),
            pltpu.SemaphoreType.DMA,
        ],
    )
    return f(dst2d, ones_i, zeros_i)


def _agg_call(y, src2d, dst2d, zeros_f):
    mesh = plsc.VectorSubcoreMesh(**_MESH)
    f = pl.kernel(
        _agg_body,
        out_type=jax.ShapeDtypeStruct((2 * NP, D), jnp.float32),
        mesh=mesh,
        scratch_types=[
            pltpu.VMEM((2 * IB, CH), jnp.int32),
            pltpu.VMEM((2 * IB, CH), jnp.int32),
            pltpu.VMEM((CH, D), jnp.float32),
            pltpu.VMEM((CH, D), jnp.float32),
            pltpu.VMEM_SHARED((NP, D), jnp.float32),
            pltpu.SemaphoreType.DMA,
            pltpu.SemaphoreType.DMA,
            pltpu.SemaphoreType.DMA,
        ],
    )
    return f(y, src2d, dst2d, zeros_f)


R = 640
G = NP // R
GN = -(-N // R)


def _dinv(d0, d1):
    deg = (d0[:, 0:1] + d1[:, 0:1] + 1).astype(jnp.float32)
    return lax.rsqrt(deg)


def _b_body(x_ref, w_ref, d0, d1, y_ref, dv_ref):
    dinv = _dinv(d0, d1)
    xw = jnp.dot(x_ref[:, :], w_ref[:, :], preferred_element_type=jnp.float32)
    y_ref[:, :] = xw * dinv
    dv_ref[:, :] = dinv


def _d_body(p0, p1, y1_ref, dv_ref, b1_ref, w2_ref, h1_ref, y2_ref):
    dinv = dv_ref[:, :]
    h1 = jnp.maximum(dinv * (p0[:, :] + p1[:, :] + y1_ref[:, :])
                     + b1_ref[:, :], 0.0)
    h1_ref[:, :] = h1
    y2_ref[:, :] = dinv * jnp.dot(h1, w2_ref[:, :],
                                  preferred_element_type=jnp.float32)


def _e_body(p0, p1, y2_ref, dv_ref, b2_ref, h1_ref, wl_ref, bl_ref, o_ref):
    dinv = dv_ref[:, :]
    h2 = jnp.maximum(dinv * (p0[:, :] + p1[:, :] + y2_ref[:, :])
                     + b2_ref[:, :], 0.0)
    o_ref[:, :] = (jnp.dot(h1_ref[:, :], wl_ref[0:D, :],
                           preferred_element_type=jnp.float32)
                   + jnp.dot(h2, wl_ref[D:2 * D, :],
                             preferred_element_type=jnp.float32)
                   + bl_ref[:, :])


_ROW = pl.BlockSpec((R, D), lambda i: (i, 0))
_ROW1 = pl.BlockSpec((R, D), lambda i: (i + G, 0))
_DEG = pl.BlockSpec((R, D), lambda i: (i, 0))
_DEG1 = pl.BlockSpec((R, D), lambda i: (i + G, 0))
_WFULL = pl.BlockSpec((D, D), lambda i: (0, 0))
_BIAS = pl.BlockSpec((1, D), lambda i: (0, 0))


_DV = pl.BlockSpec((R, 1), lambda i: (i, 0))


def _b_call(x, W1, degp):
    return pl.pallas_call(
        _b_body,
        grid=(GN,),
        in_specs=[_ROW, _WFULL, _DEG, _DEG1],
        out_specs=[_ROW, _DV],
        out_shape=[jax.ShapeDtypeStruct((N, D), jnp.float32),
                   jax.ShapeDtypeStruct((N, 1), jnp.float32)],
    )(x, W1, degp, degp)


def _d_call(p1, y1, dv, b1, W2):
    return pl.pallas_call(
        _d_body,
        grid=(GN,),
        in_specs=[_ROW, _ROW1, _ROW, _DV, _BIAS, _WFULL],
        out_specs=[_ROW, _ROW],
        out_shape=[jax.ShapeDtypeStruct((N, D), jnp.float32),
                   jax.ShapeDtypeStruct((N, D), jnp.float32)],
    )(p1, p1, y1, dv, b1, W2)


def _e_call(p2, y2, dv, b2, h1, Wl, bl):
    return pl.pallas_call(
        _e_body,
        grid=(GN,),
        in_specs=[_ROW, _ROW1, _ROW, _DV, _BIAS, _ROW,
                  pl.BlockSpec((2 * D, D), lambda i: (0, 0)), _BIAS],
        out_specs=_ROW,
        out_shape=jax.ShapeDtypeStruct((N, D), jnp.float32),
    )(p2, p2, y2, dv, b2, h1, Wl, bl)


def kernel(x, edge_index, W1, b1, W2, b2, Wl, bl):
    ei = edge_index.astype(jnp.int32)
    pad = jnp.zeros((2, EPAD - E), jnp.int32)
    ei = jnp.concatenate([ei, pad], axis=1)
    src2d = ei[0].reshape(EPAD // CH, CH)
    dst2d = ei[1].reshape(EPAD // CH, CH)
    zeros_f = jnp.zeros((RPT, D), jnp.float32)
    zeros_i = jnp.zeros((RPT, D), jnp.int32)
    ones_i = jnp.ones((CH, D), jnp.int32)

    degp = _deg_call(dst2d, ones_i, zeros_i)
    y1, dv = _b_call(x, W1, degp)
    p1 = _agg_call(y1, src2d, dst2d, zeros_f)
    h1, y2 = _d_call(p1, y1, dv, b1.reshape(1, D), W2)
    p2 = _agg_call(y2, src2d, dst2d, zeros_f)
    return _e_call(p2, y2, dv, b2.reshape(1, D), h1, Wl, bl.reshape(1, D))
